# Initial kernel scaffold; baseline (speedup 1.0000x reference)
import jax
import jax.numpy as jnp
from jax.experimental import pallas as pl


def kernel(in_feat, edge_index, W_self1, W_neigh1, b1, Ws_self, Ws_neigh, bs, gammas, betas):
    raise NotImplementedError("write your pallas kernel here")


# speedup vs baseline: 2.6324x; 2.6324x over previous
import functools

import jax
import jax.numpy as jnp
from jax import lax
from jax.experimental import pallas as pl
from jax.experimental.pallas import tpu as pltpu
from jax.experimental.pallas import tpu_sc as plsc

_N = 10000
_E = 160000
_DIN = 256
_DH = 512
_EPS = 1e-5

_NC = 2
_NS = 16
_EPAD = 163840
_ROWS = _EPAD // 128
_NPAD = 10240
_TRASH = _N
_RPT = _ROWS // _NS


def _make_mesh():
  return plsc.VectorSubcoreMesh(core_axis_name="c", subcore_axis_name="s",
                                num_cores=_NC, num_subcores=_NS)


def _make_deg(interpret=False):
  @functools.partial(
      pl.kernel,
      mesh=_make_mesh(),
      out_type=jax.ShapeDtypeStruct((_NC, _NPAD), jnp.float32),
      scratch_types=[
          pltpu.VMEM((40, 128), jnp.int32),
          pltpu.VMEM((128,), jnp.float32),
          pltpu.VMEM((640,), jnp.float32),
          pltpu.VMEM_SHARED((_NPAD,), jnp.float32),
      ],
      interpret=interpret,
  )
  def deg_kernel(dstm, out, dst_v, ones_v, zb, acc):
    cid = lax.axis_index("c")
    sid = lax.axis_index("s")
    r0 = cid * 640 + sid * 40
    pltpu.sync_copy(dstm.at[pl.ds(r0, 40)], dst_v)

    def fill(k, _):
      ones_v[pl.ds(k * 16, 16)] = jnp.ones((16,), jnp.float32)
      zb[pl.ds(k * 16, 16)] = jnp.zeros((16,), jnp.float32)
      return 0

    lax.fori_loop(0, 8, fill, 0)

    def zrest(k, _):
      zb[pl.ds(128 + k * 16, 16)] = jnp.zeros((16,), jnp.float32)
      return 0

    lax.fori_loop(0, (640 - 128) // 16, zrest, 0)
    pltpu.sync_copy(zb, acc.at[pl.ds(sid * 640, 640)])
    plsc.subcore_barrier()

    def add_body(j, _):
      pltpu.sync_copy(ones_v, acc.at[dst_v.at[j]], add=True)
      return 0

    lax.fori_loop(0, 40, add_body, 0)
    plsc.subcore_barrier()
    pltpu.sync_copy(acc.at[pl.ds(sid * 640, 640)],
                    out.at[cid, pl.ds(sid * 640, 640)])

  return deg_kernel


def _make_agg(nc, interpret=False):
  cpc = nc // _NC

  @functools.partial(
      pl.kernel,
      mesh=_make_mesh(),
      out_type=jax.ShapeDtypeStruct((nc, _NPAD, 128), jnp.float32),
      scratch_types=[
          pltpu.VMEM((_RPT, 128), jnp.int32),
          pltpu.VMEM((_RPT, 128), jnp.int32),
          pltpu.VMEM((_RPT, 128), jnp.int32),
          pltpu.VMEM((16, 128), jnp.float32),
          pltpu.VMEM((128, 128), jnp.float32),
          pltpu.VMEM_SHARED((_NPAD, 128), jnp.float32),
      ],
      interpret=interpret,
  )
  def agg_kernel(hall, srcm, dstm, out, src_v, srcg_v, dst_v, zb, gbuf, acc):
    cid = lax.axis_index("c")
    sid = lax.axis_index("s")
    r0 = sid * _RPT
    pltpu.sync_copy(srcm.at[pl.ds(r0, _RPT)], src_v)
    pltpu.sync_copy(dstm.at[pl.ds(r0, _RPT)], dst_v)

    def zfill(k, _):
      zb[k // 8, pl.ds((k % 8) * 16, 16)] = jnp.zeros((16,), jnp.float32)
      return 0

    lax.fori_loop(0, 16 * 8, zfill, 0)

    for p in range(cpc):
      g = cid * cpc + p
      off = jnp.full((16,), g * _N, jnp.int32)

      def zacc(q, _):
        pltpu.sync_copy(zb, acc.at[pl.ds(sid * 640 + q * 16, 16)])
        return 0

      lax.fori_loop(0, 640 // 16, zacc, 0)

      def shift(k, _):
        j = k // 8
        v = (k % 8) * 16
        srcg_v[j, pl.ds(v, 16)] = src_v[j, pl.ds(v, 16)] + off
        return 0

      lax.fori_loop(0, _RPT * 8, shift, 0)
      plsc.subcore_barrier()

      def acc_body(j, _):
        pltpu.sync_copy(hall.at[srcg_v.at[j]], gbuf)
        pltpu.sync_copy(gbuf, acc.at[dst_v.at[j]], add=True)
        return 0

      lax.fori_loop(0, _RPT, acc_body, 0)
      plsc.subcore_barrier()
      pltpu.sync_copy(acc.at[pl.ds(sid * 640, 640)],
                      out.at[g, pl.ds(sid * 640, 640)])
      plsc.subcore_barrier()

  return agg_kernel


_RB = 1000
_GRID = _N // _RB


def _tc_layer1(x, agg, degT, ws, wn, b):
  def body(x_ref, agg_ref, deg_ref, ws_ref, wn_ref, b_ref, yk_ref):
    z = jnp.dot(x_ref[...], ws_ref[...], preferred_element_type=jnp.float32)
    deg = deg_ref[:, 0] + deg_ref[:, 1]
    rdeg = (1.0 / jnp.maximum(deg, 1.0))[:, None]
    for c in range(2):
      z += jnp.dot(agg_ref[c] * rdeg, wn_ref[128 * c:128 * (c + 1), :],
                   preferred_element_type=jnp.float32)
    y = jnp.maximum(z + b_ref[...], 0.0)
    for c in range(4):
      yk_ref[c] = y[:, 128 * c:128 * (c + 1)]

  return pl.pallas_call(
      body,
      grid=(_GRID,),
      in_specs=[
          pl.BlockSpec((_RB, _DIN), lambda i: (i, 0)),
          pl.BlockSpec((2, _RB, 128), lambda i: (0, i, 0)),
          pl.BlockSpec((_RB, 2), lambda i: (i, 0)),
          pl.BlockSpec((_DIN, _DH), lambda i: (0, 0)),
          pl.BlockSpec((_DIN, _DH), lambda i: (0, 0)),
          pl.BlockSpec((1, _DH), lambda i: (0, 0)),
      ],
      out_specs=pl.BlockSpec((4, _RB, 128), lambda i: (0, i, 0)),
      out_shape=jax.ShapeDtypeStruct((4, _N, 128), jnp.float32),
  )(x, agg, degT, ws, wn, b)


def _tc_layer(hc, agg, degT, ws, wn, b):
  def body(hc_ref, agg_ref, deg_ref, ws_ref, wn_ref, b_ref,
           yk_ref, sum_ref, sq_ref):
    deg = deg_ref[:, 0] + deg_ref[:, 1]
    rdeg = (1.0 / jnp.maximum(deg, 1.0))[:, None]
    z = jnp.dot(hc_ref[0], ws_ref[0:128, :], preferred_element_type=jnp.float32)
    for c in range(1, 4):
      z += jnp.dot(hc_ref[c], ws_ref[128 * c:128 * (c + 1), :],
                   preferred_element_type=jnp.float32)
    for c in range(4):
      z += jnp.dot(agg_ref[c] * rdeg, wn_ref[128 * c:128 * (c + 1), :],
                   preferred_element_type=jnp.float32)
    y = jnp.maximum(z + b_ref[...], 0.0)
    for c in range(4):
      yk_ref[c] = y[:, 128 * c:128 * (c + 1)]

    @pl.when(pl.program_id(0) == 0)
    def _():
      sum_ref[...] = jnp.zeros_like(sum_ref)
      sq_ref[...] = jnp.zeros_like(sq_ref)

    sum_ref[...] += jnp.sum(y, axis=0, keepdims=True)
    sq_ref[...] += jnp.sum(y * y, axis=0, keepdims=True)

  return pl.pallas_call(
      body,
      grid=(_GRID,),
      in_specs=[
          pl.BlockSpec((4, _RB, 128), lambda i: (0, i, 0)),
          pl.BlockSpec((4, _RB, 128), lambda i: (0, i, 0)),
          pl.BlockSpec((_RB, 2), lambda i: (i, 0)),
          pl.BlockSpec((_DH, _DH), lambda i: (0, 0)),
          pl.BlockSpec((_DH, _DH), lambda i: (0, 0)),
          pl.BlockSpec((1, _DH), lambda i: (0, 0)),
      ],
      out_specs=[
          pl.BlockSpec((4, _RB, 128), lambda i: (0, i, 0)),
          pl.BlockSpec((1, _DH), lambda i: (0, 0)),
          pl.BlockSpec((1, _DH), lambda i: (0, 0)),
      ],
      out_shape=[
          jax.ShapeDtypeStruct((4, _N, 128), jnp.float32),
          jax.ShapeDtypeStruct((1, _DH), jnp.float32),
          jax.ShapeDtypeStruct((1, _DH), jnp.float32),
      ],
  )(hc, agg, degT, ws, wn, b)


def _tc_bn(yk, sums, sq, gamma, beta, full):
  def body(yk_ref, sum_ref, sq_ref, g_ref, b_ref, out_ref):
    mu = sum_ref[...] / _N
    var = sq_ref[...] / _N - mu * mu
    scale = g_ref[...] * lax.rsqrt(var + _EPS)
    shift = b_ref[...] - mu * scale
    if full:
      y = jnp.concatenate([yk_ref[c] for c in range(4)], axis=1)
      out_ref[...] = y * scale + shift
    else:
      for c in range(4):
        out_ref[c] = (yk_ref[c] * scale[:, 128 * c:128 * (c + 1)]
                      + shift[:, 128 * c:128 * (c + 1)])

  if full:
    out_spec = pl.BlockSpec((_RB, _DH), lambda i: (i, 0))
    out_shape = jax.ShapeDtypeStruct((_N, _DH), jnp.float32)
  else:
    out_spec = pl.BlockSpec((4, _RB, 128), lambda i: (0, i, 0))
    out_shape = jax.ShapeDtypeStruct((4, _N, 128), jnp.float32)

  return pl.pallas_call(
      body,
      grid=(_GRID,),
      in_specs=[
          pl.BlockSpec((4, _RB, 128), lambda i: (0, i, 0)),
          pl.BlockSpec((1, _DH), lambda i: (0, 0)),
          pl.BlockSpec((1, _DH), lambda i: (0, 0)),
          pl.BlockSpec((1, _DH), lambda i: (0, 0)),
          pl.BlockSpec((1, _DH), lambda i: (0, 0)),
      ],
      out_specs=out_spec,
      out_shape=out_shape,
  )(yk, sums, sq, gamma, beta)


def kernel(in_feat, edge_index, W_self1, W_neigh1, b1, Ws_self, Ws_neigh,
           bs, gammas, betas):
  src = edge_index[0]
  dst = edge_index[1]
  pad = _EPAD - _E
  srcm = jnp.concatenate([src, jnp.zeros((pad,), jnp.int32)]).reshape(_ROWS, 128)
  dstm = jnp.concatenate([dst, jnp.full((pad,), _TRASH, jnp.int32)]).reshape(_ROWS, 128)

  degp = _make_deg()(dstm)
  degT = degp[:, :_N].T

  xc = in_feat.reshape(_N, 2, 128).transpose(1, 0, 2)
  agg1 = _make_agg(2)(xc.reshape(2 * _N, 128), srcm, dstm)[:, :_N]

  h = _tc_layer1(in_feat, agg1, degT, W_self1, W_neigh1, b1.reshape(1, _DH))

  for i in range(2):
    agg = _make_agg(4)(h.reshape(4 * _N, 128), srcm, dstm)[:, :_N]
    yk, sums, sq = _tc_layer(h, agg, degT, Ws_self[i], Ws_neigh[i],
                             bs[i].reshape(1, _DH))
    h = _tc_bn(yk, sums, sq, gammas[i].reshape(1, _DH),
               betas[i].reshape(1, _DH), full=(i == 1))
  return h


# speedup vs baseline: 3.0345x; 1.1527x over previous
import functools

import jax
import jax.numpy as jnp
from jax import lax
from jax.experimental import pallas as pl
from jax.experimental.pallas import tpu as pltpu
from jax.experimental.pallas import tpu_sc as plsc

_N = 10000
_E = 160000
_DIN = 256
_DH = 512
_EPS = 1e-5

_NC = 2
_NS = 16
_EPAD = 163840
_ROWS = _EPAD // 128
_NPAD = 10112
_TRASH = _N
_RPT = _ROWS // _NS
_DPAD = 10240

_W = 128
_C1 = _DIN // _W
_CH = _DH // _W


def _make_mesh():
  return plsc.VectorSubcoreMesh(core_axis_name="c", subcore_axis_name="s",
                                num_cores=_NC, num_subcores=_NS)


def _make_deg(interpret=False):
  @functools.partial(
      pl.kernel,
      mesh=_make_mesh(),
      out_type=jax.ShapeDtypeStruct((_NC, _DPAD), jnp.float32),
      scratch_types=[
          pltpu.VMEM((40, 128), jnp.int32),
          pltpu.VMEM((128,), jnp.float32),
          pltpu.VMEM((640,), jnp.float32),
          pltpu.VMEM_SHARED((_DPAD,), jnp.float32),
      ],
      interpret=interpret,
  )
  def deg_kernel(dstm, out, dst_v, ones_v, zb, acc):
    cid = lax.axis_index("c")
    sid = lax.axis_index("s")
    r0 = cid * 640 + sid * 40
    pltpu.sync_copy(dstm.at[pl.ds(r0, 40)], dst_v)

    def fill(k, _):
      ones_v[pl.ds(k * 16, 16)] = jnp.ones((16,), jnp.float32)
      zb[pl.ds(k * 16, 16)] = jnp.zeros((16,), jnp.float32)
      return 0

    lax.fori_loop(0, 8, fill, 0)

    def zrest(k, _):
      zb[pl.ds(128 + k * 16, 16)] = jnp.zeros((16,), jnp.float32)
      return 0

    lax.fori_loop(0, (640 - 128) // 16, zrest, 0)
    pltpu.sync_copy(zb, acc.at[pl.ds(sid * 640, 640)])
    plsc.subcore_barrier()

    def add_body(j, _):
      pltpu.sync_copy(ones_v, acc.at[dst_v.at[j]], add=True)
      return 0

    lax.fori_loop(0, 40, add_body, 0)
    plsc.subcore_barrier()
    pltpu.sync_copy(acc.at[pl.ds(sid * 640, 640)],
                    out.at[cid, pl.ds(sid * 640, 640)])

  return deg_kernel


def _make_agg(ncol, interpret=False):
  cpc = ncol // _NC
  nbuf = 2
  lag = 1
  half = _RPT // 2

  @functools.partial(
      pl.kernel,
      mesh=_make_mesh(),
      out_type=jax.ShapeDtypeStruct((ncol, _NPAD, _W), jnp.float32),
      scratch_types=[
          pltpu.VMEM((half, 128), jnp.int32),
          pltpu.VMEM((half, 128), jnp.int32),
          pltpu.VMEM((8, _W), jnp.float32),
          pltpu.VMEM((nbuf, 128, _W), jnp.float32),
          pltpu.VMEM_SHARED((_NPAD, _W), jnp.float32),
          pltpu.SemaphoreType.DMA((nbuf,)),
          pltpu.SemaphoreType.DMA((nbuf,)),
      ],
      interpret=interpret,
  )
  def agg_kernel(hall, srcm, dstm, out, srcg_v, dst_v, zb, gbufs, acc,
                 gsems, ssems):
    cid = lax.axis_index("c")
    sid = lax.axis_index("s")
    r0 = sid * _RPT

    for zr in range(8):
      for zc in range(_W // 16):
        zb[zr, pl.ds(zc * 16, 16)] = jnp.zeros((16,), jnp.float32)

    def gather_start(j, b):
      pltpu.async_copy(hall.at[srcg_v.at[j]], gbufs.at[b], gsems.at[b])

    def gather_wait(j, b):
      pltpu.make_async_copy(hall.at[srcg_v.at[j]], gbufs.at[b],
                            gsems.at[b]).wait()

    def scatter_start(j, b):
      pltpu.async_copy(gbufs.at[b], acc.at[dst_v.at[j]], ssems.at[b],
                       add=True)

    def scatter_wait(j, b):
      pltpu.make_async_copy(gbufs.at[b], acc.at[dst_v.at[j]],
                            ssems.at[b]).wait()

    def run_half(g, h):
      pltpu.sync_copy(srcm.at[pl.ds(r0 + h * half, half)], srcg_v)
      pltpu.sync_copy(dstm.at[pl.ds(r0 + h * half, half)], dst_v)
      off = jnp.full((16,), g * _N, jnp.int32)

      def shift(k, _):
        j = k // 8
        v = (k % 8) * 16
        srcg_v[j, pl.ds(v, 16)] = srcg_v[j, pl.ds(v, 16)] + off
        return 0

      lax.fori_loop(0, half * 8, shift, 0)

      def pipe_body(j, _):
        b = lax.rem(j, nbuf)

        @pl.when(jnp.logical_and(j >= nbuf, j - nbuf < half))
        def _():
          scatter_wait(j - nbuf, b)

        @pl.when(j < half)
        def _():
          gather_start(j, b)

        @pl.when(jnp.logical_and(j >= lag, j - lag < half))
        def _():
          bl = lax.rem(j - lag, nbuf)
          gather_wait(j - lag, bl)
          scatter_start(j - lag, bl)

        return 0

      lax.fori_loop(0, half + lag, pipe_body, 0)
      for i in range(half + lag - nbuf, half):
        scatter_wait(i, i % nbuf)

    for p in range(cpc):
      g = cid * cpc + p

      def zacc(q, _):
        pltpu.sync_copy(zb, acc.at[pl.ds(sid * 632 + q * 8, 8)])
        return 0

      lax.fori_loop(0, 632 // 8, zacc, 0)
      plsc.subcore_barrier()
      run_half(g, 0)
      run_half(g, 1)
      plsc.subcore_barrier()
      pltpu.sync_copy(acc.at[pl.ds(sid * 632, 632)],
                      out.at[g, pl.ds(sid * 632, 632)])
      plsc.subcore_barrier()

  return agg_kernel


_RB = 1000
_GRID = _N // _RB


def _tc_layer1(x, agg, degT, ws, wn, b):
  def body(x_ref, agg_ref, deg_ref, ws_ref, wn_ref, b_ref, yk_ref):
    z = jnp.dot(x_ref[...], ws_ref[...], preferred_element_type=jnp.float32)
    deg = deg_ref[:, 0] + deg_ref[:, 1]
    rdeg = (1.0 / jnp.maximum(deg, 1.0))[:, None]
    for c in range(_C1):
      z += jnp.dot(agg_ref[c] * rdeg,
                   wn_ref[_W * c:_W * (c + 1), :],
                   preferred_element_type=jnp.float32)
    y = jnp.maximum(z + b_ref[...], 0.0)
    for c in range(_CH):
      yk_ref[c] = y[:, _W * c:_W * (c + 1)]

  return pl.pallas_call(
      body,
      grid=(_GRID,),
      in_specs=[
          pl.BlockSpec((_RB, _DIN), lambda i: (i, 0)),
          pl.BlockSpec((_C1, _RB, _W), lambda i: (0, i, 0)),
          pl.BlockSpec((_RB, 2), lambda i: (i, 0)),
          pl.BlockSpec((_DIN, _DH), lambda i: (0, 0)),
          pl.BlockSpec((_DIN, _DH), lambda i: (0, 0)),
          pl.BlockSpec((1, _DH), lambda i: (0, 0)),
      ],
      out_specs=pl.BlockSpec((_CH, _RB, _W), lambda i: (0, i, 0)),
      out_shape=jax.ShapeDtypeStruct((_CH, _N, _W), jnp.float32),
  )(x, agg, degT, ws, wn, b)


def _tc_layer(hc, agg, degT, ws, wn, b):
  def body(hc_ref, agg_ref, deg_ref, ws_ref, wn_ref, b_ref,
           yk_ref, sum_ref, sq_ref):
    deg = deg_ref[:, 0] + deg_ref[:, 1]
    rdeg = (1.0 / jnp.maximum(deg, 1.0))[:, None]
    z = jnp.dot(hc_ref[0], ws_ref[0:_W, :],
                preferred_element_type=jnp.float32)
    for c in range(1, _CH):
      z += jnp.dot(hc_ref[c],
                   ws_ref[_W * c:_W * (c + 1), :],
                   preferred_element_type=jnp.float32)
    for c in range(_CH):
      z += jnp.dot(agg_ref[c] * rdeg,
                   wn_ref[_W * c:_W * (c + 1), :],
                   preferred_element_type=jnp.float32)
    y = jnp.maximum(z + b_ref[...], 0.0)
    for c in range(_CH):
      yk_ref[c] = y[:, _W * c:_W * (c + 1)]

    @pl.when(pl.program_id(0) == 0)
    def _():
      sum_ref[...] = jnp.zeros_like(sum_ref)
      sq_ref[...] = jnp.zeros_like(sq_ref)

    sum_ref[...] += jnp.sum(y, axis=0, keepdims=True)
    sq_ref[...] += jnp.sum(y * y, axis=0, keepdims=True)

  return pl.pallas_call(
      body,
      grid=(_GRID,),
      in_specs=[
          pl.BlockSpec((_CH, _RB, _W), lambda i: (0, i, 0)),
          pl.BlockSpec((_CH, _RB, _W), lambda i: (0, i, 0)),
          pl.BlockSpec((_RB, 2), lambda i: (i, 0)),
          pl.BlockSpec((_DH, _DH), lambda i: (0, 0)),
          pl.BlockSpec((_DH, _DH), lambda i: (0, 0)),
          pl.BlockSpec((1, _DH), lambda i: (0, 0)),
      ],
      out_specs=[
          pl.BlockSpec((_CH, _RB, _W), lambda i: (0, i, 0)),
          pl.BlockSpec((1, _DH), lambda i: (0, 0)),
          pl.BlockSpec((1, _DH), lambda i: (0, 0)),
      ],
      out_shape=[
          jax.ShapeDtypeStruct((_CH, _N, _W), jnp.float32),
          jax.ShapeDtypeStruct((1, _DH), jnp.float32),
          jax.ShapeDtypeStruct((1, _DH), jnp.float32),
      ],
  )(hc, agg, degT, ws, wn, b)


def _tc_bn(yk, sums, sq, gamma, beta, full):
  def body(yk_ref, sum_ref, sq_ref, g_ref, b_ref, out_ref):
    mu = sum_ref[...] / _N
    var = sq_ref[...] / _N - mu * mu
    scale = g_ref[...] * lax.rsqrt(var + _EPS)
    shift = b_ref[...] - mu * scale
    if full:
      y = jnp.concatenate([yk_ref[c] for c in range(_CH)], axis=1)
      out_ref[...] = y * scale + shift
    else:
      for c in range(_CH):
        out_ref[c] = (yk_ref[c] * scale[:, _W * c:_W * (c + 1)]
                      + shift[:, _W * c:_W * (c + 1)])

  if full:
    out_spec = pl.BlockSpec((_RB, _DH), lambda i: (i, 0))
    out_shape = jax.ShapeDtypeStruct((_N, _DH), jnp.float32)
  else:
    out_spec = pl.BlockSpec((_CH, _RB, _W), lambda i: (0, i, 0))
    out_shape = jax.ShapeDtypeStruct((_CH, _N, _W), jnp.float32)

  return pl.pallas_call(
      body,
      grid=(_GRID,),
      in_specs=[
          pl.BlockSpec((_CH, _RB, _W), lambda i: (0, i, 0)),
          pl.BlockSpec((1, _DH), lambda i: (0, 0)),
          pl.BlockSpec((1, _DH), lambda i: (0, 0)),
          pl.BlockSpec((1, _DH), lambda i: (0, 0)),
          pl.BlockSpec((1, _DH), lambda i: (0, 0)),
      ],
      out_specs=out_spec,
      out_shape=out_shape,
  )(yk, sums, sq, gamma, beta)


def kernel(in_feat, edge_index, W_self1, W_neigh1, b1, Ws_self, Ws_neigh,
           bs, gammas, betas):
  src = edge_index[0]
  dst = edge_index[1]
  pad = _EPAD - _E
  srcm = jnp.concatenate([src, jnp.zeros((pad,), jnp.int32)]).reshape(_ROWS, 128)
  dstm = jnp.concatenate([dst, jnp.full((pad,), _TRASH, jnp.int32)]).reshape(_ROWS, 128)

  degp = _make_deg()(dstm)
  degT = degp[:, :_N].T

  xc = in_feat.reshape(_N, _C1, _W).transpose(1, 0, 2)
  agg1 = _make_agg(_C1)(xc.reshape(_C1 * _N, _W), srcm, dstm)[:, :_N]

  h = _tc_layer1(in_feat, agg1, degT, W_self1, W_neigh1, b1.reshape(1, _DH))

  for i in range(2):
    agg = _make_agg(_CH)(h.reshape(_CH * _N, _W), srcm, dstm)[:, :_N]
    yk, sums, sq = _tc_layer(h, agg, degT, Ws_self[i], Ws_neigh[i],
                             bs[i].reshape(1, _DH))
    h = _tc_bn(yk, sums, sq, gammas[i].reshape(1, _DH),
               betas[i].reshape(1, _DH), full=(i == 1))
  return h


# speedup vs baseline: 3.0888x; 1.0179x over previous
import functools

import jax
import jax.numpy as jnp
from jax import lax
from jax.experimental import pallas as pl
from jax.experimental.pallas import tpu as pltpu
from jax.experimental.pallas import tpu_sc as plsc

_N = 10000
_E = 160000
_DIN = 256
_DH = 512
_EPS = 1e-5

_NC = 2
_NS = 16
_EPAD = 163840
_ROWS = _EPAD // 128
_NPAD = 10112
_TRASH = _N
_RPT = _ROWS // _NS
_DPAD = 10240

_W = 128
_C1 = _DIN // _W
_CH = _DH // _W


def _make_mesh():
  return plsc.VectorSubcoreMesh(core_axis_name="c", subcore_axis_name="s",
                                num_cores=_NC, num_subcores=_NS)


def _make_deg(interpret=False):
  @functools.partial(
      pl.kernel,
      mesh=_make_mesh(),
      out_type=jax.ShapeDtypeStruct((_NC, _DPAD), jnp.float32),
      scratch_types=[
          pltpu.VMEM((40, 128), jnp.int32),
          pltpu.VMEM((128,), jnp.float32),
          pltpu.VMEM((640,), jnp.float32),
          pltpu.VMEM_SHARED((_DPAD,), jnp.float32),
      ],
      interpret=interpret,
  )
  def deg_kernel(dstm, out, dst_v, ones_v, zb, acc):
    cid = lax.axis_index("c")
    sid = lax.axis_index("s")
    r0 = cid * 640 + sid * 40
    pltpu.sync_copy(dstm.at[pl.ds(r0, 40)], dst_v)

    def fill(k, _):
      ones_v[pl.ds(k * 16, 16)] = jnp.ones((16,), jnp.float32)
      zb[pl.ds(k * 16, 16)] = jnp.zeros((16,), jnp.float32)
      return 0

    lax.fori_loop(0, 8, fill, 0)

    def zrest(k, _):
      zb[pl.ds(128 + k * 16, 16)] = jnp.zeros((16,), jnp.float32)
      return 0

    lax.fori_loop(0, (640 - 128) // 16, zrest, 0)
    pltpu.sync_copy(zb, acc.at[pl.ds(sid * 640, 640)])
    plsc.subcore_barrier()

    def add_body(j, _):
      pltpu.sync_copy(ones_v, acc.at[dst_v.at[j]], add=True)
      return 0

    lax.fori_loop(0, 40, add_body, 0)
    plsc.subcore_barrier()
    pltpu.sync_copy(acc.at[pl.ds(sid * 640, 640)],
                    out.at[cid, pl.ds(sid * 640, 640)])

  return deg_kernel


def _make_agg(ncol, interpret=False):
  cpc = ncol // _NC
  nbuf = 2
  lag = 1
  half = _RPT // 2

  @functools.partial(
      pl.kernel,
      mesh=_make_mesh(),
      out_type=jax.ShapeDtypeStruct((ncol, _NPAD, _W), jnp.float32),
      scratch_types=[
          pltpu.VMEM((half, 128), jnp.int32),
          pltpu.VMEM((half, 128), jnp.int32),
          pltpu.VMEM((8, _W), jnp.float32),
          pltpu.VMEM((nbuf, 128, _W), jnp.float32),
          pltpu.VMEM_SHARED((_NPAD, _W), jnp.float32),
          pltpu.SemaphoreType.DMA((nbuf,)),
          pltpu.SemaphoreType.DMA((nbuf,)),
          pltpu.SemaphoreType.DMA,
      ],
      interpret=interpret,
  )
  def agg_kernel(hall, srcm, dstm, out, srcg_v, dst_v, zb, gbufs, acc,
                 gsems, ssems, zsem):
    cid = lax.axis_index("c")
    sid = lax.axis_index("s")
    r0 = sid * _RPT

    for zr in range(8):
      for zc in range(_W // 16):
        zb[zr, pl.ds(zc * 16, 16)] = jnp.zeros((16,), jnp.float32)

    def gather_start(j, b):
      pltpu.async_copy(hall.at[srcg_v.at[j]], gbufs.at[b], gsems.at[b])

    def gather_wait(j, b):
      pltpu.make_async_copy(hall.at[srcg_v.at[j]], gbufs.at[b],
                            gsems.at[b]).wait()

    def scatter_start(j, b):
      pltpu.async_copy(gbufs.at[b], acc.at[dst_v.at[j]], ssems.at[b],
                       add=True)

    def scatter_wait(j, b):
      pltpu.make_async_copy(gbufs.at[b], acc.at[dst_v.at[j]],
                            ssems.at[b]).wait()

    def run_half(g, h):
      pltpu.sync_copy(srcm.at[pl.ds(r0 + h * half, half)], srcg_v)
      pltpu.sync_copy(dstm.at[pl.ds(r0 + h * half, half)], dst_v)
      off = jnp.full((16,), g * _N, jnp.int32)

      def shift(k, _):
        j = k // 8
        v = (k % 8) * 16
        srcg_v[j, pl.ds(v, 16)] = srcg_v[j, pl.ds(v, 16)] + off
        return 0

      lax.fori_loop(0, half * 8, shift, 0)

      def pipe_body(j, _):
        b = lax.rem(j, nbuf)

        @pl.when(jnp.logical_and(j >= nbuf, j - nbuf < half))
        def _():
          scatter_wait(j - nbuf, b)

        @pl.when(j < half)
        def _():
          gather_start(j, b)

        @pl.when(jnp.logical_and(j >= lag, j - lag < half))
        def _():
          bl = lax.rem(j - lag, nbuf)
          gather_wait(j - lag, bl)
          scatter_start(j - lag, bl)

        return 0

      lax.fori_loop(0, half + lag, pipe_body, 0)
      for i in range(half + lag - nbuf, half):
        scatter_wait(i, i % nbuf)

    for p in range(cpc):
      g = cid * cpc + p

      def zacc(q, _):
        pltpu.async_copy(zb, acc.at[pl.ds(sid * 632 + q * 8, 8)], zsem)
        return 0

      lax.fori_loop(0, 632 // 8, zacc, 0)

      def zdrain(q, _):
        pltpu.make_async_copy(zb, acc.at[pl.ds(sid * 632 + q * 8, 8)],
                              zsem).wait()
        return 0

      lax.fori_loop(0, 632 // 8, zdrain, 0)
      plsc.subcore_barrier()
      run_half(g, 0)
      run_half(g, 1)
      plsc.subcore_barrier()
      pltpu.sync_copy(acc.at[pl.ds(sid * 632, 632)],
                      out.at[g, pl.ds(sid * 632, 632)])
      plsc.subcore_barrier()

  return agg_kernel


_RB = 1000
_GRID = _N // _RB


def _tc_layer1(x, agg, degT, ws, wn, b):
  def body(x_ref, agg_ref, deg_ref, ws_ref, wn_ref, b_ref, yk_ref):
    z = jnp.dot(x_ref[...], ws_ref[...], preferred_element_type=jnp.float32)
    deg = deg_ref[:, 0] + deg_ref[:, 1]
    rdeg = (1.0 / jnp.maximum(deg, 1.0))[:, None]
    for c in range(_C1):
      z += jnp.dot(agg_ref[c] * rdeg,
                   wn_ref[_W * c:_W * (c + 1), :],
                   preferred_element_type=jnp.float32)
    y = jnp.maximum(z + b_ref[...], 0.0)
    for c in range(_CH):
      yk_ref[c] = y[:, _W * c:_W * (c + 1)]

  return pl.pallas_call(
      body,
      grid=(_GRID,),
      in_specs=[
          pl.BlockSpec((_RB, _DIN), lambda i: (i, 0)),
          pl.BlockSpec((_C1, _RB, _W), lambda i: (0, i, 0)),
          pl.BlockSpec((_RB, 2), lambda i: (i, 0)),
          pl.BlockSpec((_DIN, _DH), lambda i: (0, 0)),
          pl.BlockSpec((_DIN, _DH), lambda i: (0, 0)),
          pl.BlockSpec((1, _DH), lambda i: (0, 0)),
      ],
      out_specs=pl.BlockSpec((_CH, _RB, _W), lambda i: (0, i, 0)),
      out_shape=jax.ShapeDtypeStruct((_CH, _N, _W), jnp.float32),
  )(x, agg, degT, ws, wn, b)


def _tc_layer(hc, agg, degT, ws, wn, b):
  def body(hc_ref, agg_ref, deg_ref, ws_ref, wn_ref, b_ref,
           yk_ref, sum_ref, sq_ref):
    deg = deg_ref[:, 0] + deg_ref[:, 1]
    rdeg = (1.0 / jnp.maximum(deg, 1.0))[:, None]
    z = jnp.dot(hc_ref[0], ws_ref[0:_W, :],
                preferred_element_type=jnp.float32)
    for c in range(1, _CH):
      z += jnp.dot(hc_ref[c],
                   ws_ref[_W * c:_W * (c + 1), :],
                   preferred_element_type=jnp.float32)
    for c in range(_CH):
      z += jnp.dot(agg_ref[c] * rdeg,
                   wn_ref[_W * c:_W * (c + 1), :],
                   preferred_element_type=jnp.float32)
    y = jnp.maximum(z + b_ref[...], 0.0)
    for c in range(_CH):
      yk_ref[c] = y[:, _W * c:_W * (c + 1)]

    @pl.when(pl.program_id(0) == 0)
    def _():
      sum_ref[...] = jnp.zeros_like(sum_ref)
      sq_ref[...] = jnp.zeros_like(sq_ref)

    sum_ref[...] += jnp.sum(y, axis=0, keepdims=True)
    sq_ref[...] += jnp.sum(y * y, axis=0, keepdims=True)

  return pl.pallas_call(
      body,
      grid=(_GRID,),
      in_specs=[
          pl.BlockSpec((_CH, _RB, _W), lambda i: (0, i, 0)),
          pl.BlockSpec((_CH, _RB, _W), lambda i: (0, i, 0)),
          pl.BlockSpec((_RB, 2), lambda i: (i, 0)),
          pl.BlockSpec((_DH, _DH), lambda i: (0, 0)),
          pl.BlockSpec((_DH, _DH), lambda i: (0, 0)),
          pl.BlockSpec((1, _DH), lambda i: (0, 0)),
      ],
      out_specs=[
          pl.BlockSpec((_CH, _RB, _W), lambda i: (0, i, 0)),
          pl.BlockSpec((1, _DH), lambda i: (0, 0)),
          pl.BlockSpec((1, _DH), lambda i: (0, 0)),
      ],
      out_shape=[
          jax.ShapeDtypeStruct((_CH, _N, _W), jnp.float32),
          jax.ShapeDtypeStruct((1, _DH), jnp.float32),
          jax.ShapeDtypeStruct((1, _DH), jnp.float32),
      ],
  )(hc, agg, degT, ws, wn, b)


def _tc_bn(yk, sums, sq, gamma, beta, full):
  def body(yk_ref, sum_ref, sq_ref, g_ref, b_ref, out_ref):
    mu = sum_ref[...] / _N
    var = sq_ref[...] / _N - mu * mu
    scale = g_ref[...] * lax.rsqrt(var + _EPS)
    shift = b_ref[...] - mu * scale
    if full:
      y = jnp.concatenate([yk_ref[c] for c in range(_CH)], axis=1)
      out_ref[...] = y * scale + shift
    else:
      for c in range(_CH):
        out_ref[c] = (yk_ref[c] * scale[:, _W * c:_W * (c + 1)]
                      + shift[:, _W * c:_W * (c + 1)])

  if full:
    out_spec = pl.BlockSpec((_RB, _DH), lambda i: (i, 0))
    out_shape = jax.ShapeDtypeStruct((_N, _DH), jnp.float32)
  else:
    out_spec = pl.BlockSpec((_CH, _RB, _W), lambda i: (0, i, 0))
    out_shape = jax.ShapeDtypeStruct((_CH, _N, _W), jnp.float32)

  return pl.pallas_call(
      body,
      grid=(_GRID,),
      in_specs=[
          pl.BlockSpec((_CH, _RB, _W), lambda i: (0, i, 0)),
          pl.BlockSpec((1, _DH), lambda i: (0, 0)),
          pl.BlockSpec((1, _DH), lambda i: (0, 0)),
          pl.BlockSpec((1, _DH), lambda i: (0, 0)),
          pl.BlockSpec((1, _DH), lambda i: (0, 0)),
      ],
      out_specs=out_spec,
      out_shape=out_shape,
  )(yk, sums, sq, gamma, beta)


def kernel(in_feat, edge_index, W_self1, W_neigh1, b1, Ws_self, Ws_neigh,
           bs, gammas, betas):
  src = edge_index[0]
  dst = edge_index[1]
  pad = _EPAD - _E
  srcm = jnp.concatenate([src, jnp.zeros((pad,), jnp.int32)]).reshape(_ROWS, 128)
  dstm = jnp.concatenate([dst, jnp.full((pad,), _TRASH, jnp.int32)]).reshape(_ROWS, 128)

  degp = _make_deg()(dstm)
  degT = degp[:, :_N].T

  xc = in_feat.reshape(_N, _C1, _W).transpose(1, 0, 2)
  agg1 = _make_agg(_C1)(xc.reshape(_C1 * _N, _W), srcm, dstm)[:, :_N]

  h = _tc_layer1(in_feat, agg1, degT, W_self1, W_neigh1, b1.reshape(1, _DH))

  for i in range(2):
    agg = _make_agg(_CH)(h.reshape(_CH * _N, _W), srcm, dstm)[:, :_N]
    yk, sums, sq = _tc_layer(h, agg, degT, Ws_self[i], Ws_neigh[i],
                             bs[i].reshape(1, _DH))
    h = _tc_bn(yk, sums, sq, gammas[i].reshape(1, _DH),
               betas[i].reshape(1, _DH), full=(i == 1))
  return h
